# speedup vs baseline: 10.1614x; 10.1614x over previous
import functools

import jax
import jax.numpy as jnp
from jax import lax
from jax.experimental import pallas as pl
from jax.experimental.pallas import tpu as pltpu
from jax.experimental.pallas import tpu_sc as plsc

N = 10000
D = 128
E = 320000
NP = 10240
NTILES = 32
CW = 128
CH = 79
EPT = CH * CW
EP = NTILES * EPT
ROWS_PER_TILE = NP // 16
BLK = 640
GRID = NP // BLK
EPS = 1e-5
FN = float(N)

_mesh = plsc.VectorSubcoreMesh(core_axis_name="c", subcore_axis_name="s")


@functools.partial(
    pl.kernel,
    out_type=jax.ShapeDtypeStruct((2, NP), jnp.float32),
    mesh=_mesh,
    scratch_types=[
        pltpu.VMEM((CH, CW), jnp.int32),
        pltpu.VMEM((CW,), jnp.float32),
        pltpu.VMEM((ROWS_PER_TILE,), jnp.float32),
        pltpu.VMEM_SHARED((NP,), jnp.float32),
    ],
)
def _sc_degree(dst3_hbm, out_hbm, idx_v, ones_v, zbuf_v, acc_s):
    cid = lax.axis_index("c")
    sid = lax.axis_index("s")
    wid = cid * 16 + sid

    @pl.loop(0, CW, step=16)
    def _(i):
        ones_v[pl.ds(i, 16)] = jnp.full((16,), 1.0, dtype=jnp.float32)

    @pl.loop(0, ROWS_PER_TILE, step=16)
    def _(i):
        zbuf_v[pl.ds(i, 16)] = jnp.full((16,), 0.0, dtype=jnp.float32)

    pltpu.sync_copy(zbuf_v, acc_s.at[pl.ds(sid * ROWS_PER_TILE, ROWS_PER_TILE)])
    plsc.subcore_barrier()

    pltpu.sync_copy(dst3_hbm.at[wid], idx_v)

    @pl.loop(0, CH)
    def _(j):
        pltpu.sync_copy(ones_v, acc_s.at[idx_v.at[j]], add=True)

    plsc.subcore_barrier()
    pltpu.sync_copy(acc_s.at[pl.ds(sid * ROWS_PER_TILE, ROWS_PER_TILE)],
                    out_hbm.at[cid, pl.ds(sid * ROWS_PER_TILE, ROWS_PER_TILE)])


@functools.partial(
    pl.kernel,
    out_type=jax.ShapeDtypeStruct((2, NP, D), jnp.float32),
    mesh=_mesh,
    scratch_types=[
        pltpu.VMEM((CH, CW), jnp.int32),
        pltpu.VMEM((CH, CW), jnp.int32),
        pltpu.VMEM((CW, D), jnp.float32),
        pltpu.VMEM_SHARED((NP, D), jnp.float32),
    ],
)
def _sc_aggregate(src3_hbm, dst3_hbm, y_hbm, zeros_hbm, out_hbm,
                  src_v, dst_v, rows_v, acc_s):
    cid = lax.axis_index("c")
    sid = lax.axis_index("s")
    wid = cid * 16 + sid

    pltpu.sync_copy(zeros_hbm, acc_s.at[pl.ds(sid * ROWS_PER_TILE, ROWS_PER_TILE)])
    pltpu.sync_copy(src3_hbm.at[wid], src_v)
    pltpu.sync_copy(dst3_hbm.at[wid], dst_v)
    plsc.subcore_barrier()

    @pl.loop(0, CH)
    def _(j):
        pltpu.sync_copy(y_hbm.at[src_v.at[j]], rows_v)
        pltpu.sync_copy(rows_v, acc_s.at[dst_v.at[j]], add=True)

    plsc.subcore_barrier()
    pltpu.sync_copy(acc_s.at[pl.ds(sid * ROWS_PER_TILE, ROWS_PER_TILE)],
                    out_hbm.at[cid, pl.ds(sid * ROWS_PER_TILE, ROWS_PER_TILE)])


def _scale_body(p0_ref, p1_ref, x_ref, w_ref, y_ref, d_ref):
    dinv = lax.rsqrt(p0_ref[...] + p1_ref[...] + 1.0)
    y_ref[...] = jnp.dot(x_ref[...], w_ref[...],
                         preferred_element_type=jnp.float32) * dinv
    d_ref[...] = dinv


def _tc_first_matmul(p0, p1, x_pad, w):
    return pl.pallas_call(
        _scale_body,
        grid=(GRID,),
        in_specs=[
            pl.BlockSpec((BLK, 1), lambda i: (i, 0)),
            pl.BlockSpec((BLK, 1), lambda i: (i, 0)),
            pl.BlockSpec((BLK, D), lambda i: (i, 0)),
            pl.BlockSpec((D, D), lambda i: (0, 0)),
        ],
        out_specs=[
            pl.BlockSpec((BLK, D), lambda i: (i, 0)),
            pl.BlockSpec((BLK, 1), lambda i: (i, 0)),
        ],
        out_shape=[
            jax.ShapeDtypeStruct((NP, D), jnp.float32),
            jax.ShapeDtypeStruct((NP, 1), jnp.float32),
        ],
    )(p0, p1, x_pad, w)


def _stats_body(p0_ref, p1_ref, y_ref, d_ref, b_ref, h_ref, s_ref):
    i = pl.program_id(0)
    h = (p0_ref[...] + p1_ref[...] + y_ref[...]) * d_ref[...] + b_ref[...]
    h_ref[...] = h
    rows = i * BLK + lax.broadcasted_iota(jnp.int32, (BLK, 1), 0)
    hm = jnp.where(rows < N, h, 0.0)
    s = jnp.concatenate([jnp.sum(hm, axis=0, keepdims=True),
                         jnp.sum(hm * hm, axis=0, keepdims=True)], axis=0)

    @pl.when(i == 0)
    def _():
        s_ref[...] = jnp.zeros_like(s_ref)

    s_ref[...] += s


def _tc_combine_stats(p0, p1, y, dinv, b):
    return pl.pallas_call(
        _stats_body,
        grid=(GRID,),
        in_specs=[
            pl.BlockSpec((BLK, D), lambda i: (i, 0)),
            pl.BlockSpec((BLK, D), lambda i: (i, 0)),
            pl.BlockSpec((BLK, D), lambda i: (i, 0)),
            pl.BlockSpec((BLK, 1), lambda i: (i, 0)),
            pl.BlockSpec((1, D), lambda i: (0, 0)),
        ],
        out_specs=[
            pl.BlockSpec((BLK, D), lambda i: (i, 0)),
            pl.BlockSpec((2, D), lambda i: (0, 0)),
        ],
        out_shape=[
            jax.ShapeDtypeStruct((NP, D), jnp.float32),
            jax.ShapeDtypeStruct((2, D), jnp.float32),
        ],
    )(p0, p1, y, dinv, b)


def _bn_matmul_body(h_ref, s_ref, g_ref, be_ref, w_ref, d_ref, y_ref):
    i = pl.program_id(0)
    mu = s_ref[0:1, :] / FN
    var = s_ref[1:2, :] / FN - mu * mu
    h = (h_ref[...] - mu) * lax.rsqrt(var + EPS) * g_ref[...] + be_ref[...]
    h = jnp.maximum(h, 0.0)
    rows = i * BLK + lax.broadcasted_iota(jnp.int32, (BLK, 1), 0)
    h = jnp.where(rows < N, h, 0.0)
    y_ref[...] = jnp.dot(h, w_ref[...],
                         preferred_element_type=jnp.float32) * d_ref[...]


def _tc_bn_matmul(h_pre, stats, g, be, w, dinv):
    return pl.pallas_call(
        _bn_matmul_body,
        grid=(GRID,),
        in_specs=[
            pl.BlockSpec((BLK, D), lambda i: (i, 0)),
            pl.BlockSpec((2, D), lambda i: (0, 0)),
            pl.BlockSpec((1, D), lambda i: (0, 0)),
            pl.BlockSpec((1, D), lambda i: (0, 0)),
            pl.BlockSpec((D, D), lambda i: (0, 0)),
            pl.BlockSpec((BLK, 1), lambda i: (i, 0)),
        ],
        out_specs=pl.BlockSpec((BLK, D), lambda i: (i, 0)),
        out_shape=jax.ShapeDtypeStruct((NP, D), jnp.float32),
    )(h_pre, stats, g, be, w, dinv)


def _bn_mean_body(h_ref, s_ref, g_ref, be_ref, m_ref):
    i = pl.program_id(0)
    mu = s_ref[0:1, :] / FN
    var = s_ref[1:2, :] / FN - mu * mu
    h = (h_ref[...] - mu) * lax.rsqrt(var + EPS) * g_ref[...] + be_ref[...]
    h = jnp.maximum(h, 0.0)
    rows = i * BLK + lax.broadcasted_iota(jnp.int32, (BLK, 1), 0)
    h = jnp.where(rows < N, h, 0.0)

    @pl.when(i == 0)
    def _():
        m_ref[...] = jnp.zeros_like(m_ref)

    m_ref[...] += jnp.sum(h, axis=0, keepdims=True)


def _tc_bn_mean(h_pre, stats, g, be):
    return pl.pallas_call(
        _bn_mean_body,
        grid=(GRID,),
        in_specs=[
            pl.BlockSpec((BLK, D), lambda i: (i, 0)),
            pl.BlockSpec((2, D), lambda i: (0, 0)),
            pl.BlockSpec((1, D), lambda i: (0, 0)),
            pl.BlockSpec((1, D), lambda i: (0, 0)),
        ],
        out_specs=pl.BlockSpec((1, D), lambda i: (0, 0)),
        out_shape=jax.ShapeDtypeStruct((1, D), jnp.float32),
    )(h_pre, stats, g, be)


def _head_body(m_ref, w1_ref, b1_ref, w2_ref, b2_ref, w3_ref, b3_ref,
               w4_ref, b4_ref, o_ref):
    h = jnp.broadcast_to(m_ref[...] / FN, (8, D))
    h = jnp.maximum(jnp.dot(h, w1_ref[...], preferred_element_type=jnp.float32)
                    + b1_ref[...], 0.0)
    h = jnp.maximum(jnp.dot(h, w2_ref[...], preferred_element_type=jnp.float32)
                    + b2_ref[...], 0.0)
    h = jnp.maximum(jnp.dot(h, w3_ref[...], preferred_element_type=jnp.float32)
                    + b3_ref[...], 0.0)
    h = jnp.dot(h, w4_ref[...], preferred_element_type=jnp.float32) + b4_ref[...]
    o_ref[...] = h[0:1, :]


def _tc_head(msum, fw1, fb1, fw2, fb2, fw3, fb3, fw4, fb4):
    return pl.pallas_call(
        _head_body,
        out_shape=jax.ShapeDtypeStruct((1, 1), jnp.float32),
    )(msum, fw1, fb1, fw2, fb2, fw3, fb3, fw4, fb4)


def kernel(x, edge_index, W1, b1, W2, b2, W3, b3, g1, be1, g2, be2, g3, be3,
           fw1, fb1, fw2, fb2, fw3, fb3, fw4, fb4):
    x_pad = jnp.zeros((NP, D), jnp.float32).at[:N].set(x)
    src = jnp.full((EP,), N, jnp.int32).at[:E].set(edge_index[0])
    dst = jnp.full((EP,), N, jnp.int32).at[:E].set(edge_index[1])
    src3 = src.reshape(NTILES, CH, CW)
    dst3 = dst.reshape(NTILES, CH, CW)
    zeros2d = jnp.zeros((ROWS_PER_TILE, D), jnp.float32)

    degp = _sc_degree(dst3)
    p0 = degp[0].reshape(NP, 1)
    p1 = degp[1].reshape(NP, 1)

    y1, dinv = _tc_first_matmul(p0, p1, x_pad, W1)

    def layer(y, b):
        agg = _sc_aggregate(src3, dst3, y, zeros2d)
        h_pre, stats = _tc_combine_stats(agg[0], agg[1], y, dinv,
                                         b.reshape(1, D))
        return h_pre, stats

    h1, s1 = layer(y1, b1)
    y2 = _tc_bn_matmul(h1, s1, g1.reshape(1, D), be1.reshape(1, D), W2, dinv)
    h2, s2 = layer(y2, b2)
    y3 = _tc_bn_matmul(h2, s2, g2.reshape(1, D), be2.reshape(1, D), W3, dinv)
    h3, s3 = layer(y3, b3)
    msum = _tc_bn_mean(h3, s3, g3.reshape(1, D), be3.reshape(1, D))

    out = _tc_head(msum, fw1, fb1.reshape(1, 128), fw2, fb2.reshape(1, 64),
                   fw3, fb3.reshape(1, 32), fw4, fb4.reshape(1, 1))
    return out.reshape(1)
